# baseline (speedup 1.0000x reference)
import functools

import jax
import jax.numpy as jnp
from jax import lax
from jax.experimental import pallas as pl
from jax.experimental.pallas import tpu as pltpu
from jax.experimental.pallas import tpu_sc as plsc

T = 2048
D = 768
F = 2048
E = 8
K = 2

B = 256
P = T * K
P_PAD = P + E * B
G = P_PAD // B

NC = 2
NS = 16
NW = NC * NS
TPW = T // NW


def _sc_mesh():
    return plsc.VectorSubcoreMesh(core_axis_name="c", subcore_axis_name="s",
                                  num_cores=NC, num_subcores=NS)


@functools.lru_cache(maxsize=None)
def _make_sc_dispatch():
    @functools.partial(
        pl.kernel,
        mesh=_sc_mesh(),
        out_type=jax.ShapeDtypeStruct((P_PAD, D), jnp.float32),
        scratch_types=[
            pltpu.VMEM((TPW,), jnp.int32),
            pltpu.VMEM((TPW,), jnp.int32),
            pltpu.VMEM((TPW, D), jnp.float32),
            pltpu.SemaphoreType.DMA,
        ],
    )
    def dispatch_kernel(x_hbm, pe_hbm, po_hbm, out_hbm, idxe_v, idxo_v,
                        rows_v, sem):
        wid = lax.axis_index("s") * NC + lax.axis_index("c")
        tb = wid * TPW
        pltpu.sync_copy(x_hbm.at[pl.ds(tb, TPW)], rows_v)
        pltpu.sync_copy(pe_hbm.at[pl.ds(tb, TPW)], idxe_v)
        pltpu.sync_copy(po_hbm.at[pl.ds(tb, TPW)], idxo_v)
        c1 = pltpu.async_copy(rows_v, out_hbm.at[idxe_v], sem)
        c2 = pltpu.async_copy(rows_v, out_hbm.at[idxo_v], sem)
        c1.wait()
        c2.wait()

    return dispatch_kernel


@functools.lru_cache(maxsize=None)
def _make_sc_combine_tkd(chunk: int):
    n_per_w = T // NW
    n_chunks = n_per_w // chunk
    assert n_per_w % chunk == 0 and chunk % 8 == 0

    @functools.partial(
        pl.kernel,
        mesh=_sc_mesh(),
        out_type=jax.ShapeDtypeStruct((T, K, D), jnp.float32),
        scratch_types=[
            pltpu.VMEM((chunk,), jnp.int32),
            pltpu.VMEM((chunk,), jnp.int32),
            pltpu.VMEM((chunk, D), jnp.float32),
            pltpu.VMEM((chunk, D), jnp.float32),
            pltpu.SemaphoreType.DMA,
        ],
    )
    def combine_kernel(table_hbm, pe_hbm, po_hbm, out_hbm, idxe_v, idxo_v,
                       rows_a, rows_b, sem):
        wid = lax.axis_index("s") * NC + lax.axis_index("c")
        base = wid * n_per_w
        for c in range(n_chunks):
            tb = base + c * chunk
            pltpu.sync_copy(pe_hbm.at[pl.ds(tb, chunk)], idxe_v)
            pltpu.sync_copy(po_hbm.at[pl.ds(tb, chunk)], idxo_v)
            c1 = pltpu.async_copy(table_hbm.at[idxe_v], rows_a, sem)
            c2 = pltpu.async_copy(table_hbm.at[idxo_v], rows_b, sem)
            c1.wait()
            c2.wait()
            pltpu.sync_copy(rows_a, out_hbm.at[pl.ds(tb, chunk), 0])
            pltpu.sync_copy(rows_b, out_hbm.at[pl.ds(tb, chunk), 1])

    return combine_kernel


@functools.lru_cache(maxsize=None)
def _make_sc_combine(n_rows: int, chunk: int):
    n_per_w = n_rows // NW
    n_chunks = n_per_w // chunk
    assert n_per_w % chunk == 0 and chunk % 8 == 0

    @functools.partial(
        pl.kernel,
        mesh=_sc_mesh(),
        out_type=jax.ShapeDtypeStruct((n_rows, D), jnp.float32),
        scratch_types=[
            pltpu.VMEM((chunk,), jnp.int32),
            pltpu.VMEM((chunk, D), jnp.float32),
            pltpu.SemaphoreType.DMA,
        ],
    )
    def gather_kernel(table_hbm, idx_hbm, out_hbm, idx_v, rows_v, sem):
        wid = lax.axis_index("s") * NC + lax.axis_index("c")
        base = wid * n_per_w
        for c in range(n_chunks):
            off = base + c * chunk
            pltpu.sync_copy(idx_hbm.at[pl.ds(off, chunk)], idx_v)
            pltpu.async_copy(table_hbm.at[idx_v], rows_v, sem).wait()
            pltpu.sync_copy(rows_v, out_hbm.at[pl.ds(off, chunk)])

    return gather_kernel


NF = 4
FB = F // NF


def _ffn_body(blk_ref, eid_ref, x_ref, w0_ref, w1_ref, w2_ref, o_ref):
    s = pl.program_id(0)
    f = pl.program_id(1)

    @pl.when(jnp.logical_and(blk_ref[s] == s, f == 0))
    def _():
        o_ref[...] = jnp.zeros_like(o_ref)

    @pl.when(blk_ref[s] == s)
    def _():
        xb = x_ref[...]
        a = jnp.dot(xb, w0_ref[0], preferred_element_type=jnp.float32,
                    precision=lax.Precision.DEFAULT)
        b = jnp.dot(xb, w1_ref[0], preferred_element_type=jnp.float32,
                    precision=lax.Precision.DEFAULT)
        h = (a * jax.nn.sigmoid(a)) * b
        o_ref[...] += jnp.dot(h, w2_ref[0],
                              preferred_element_type=jnp.float32,
                              precision=lax.Precision.DEFAULT)


_ffn_grid_spec = pltpu.PrefetchScalarGridSpec(
    num_scalar_prefetch=2,
    grid=(G, NF),
    in_specs=[
        pl.BlockSpec((B, D), lambda s, f, blk, eid: (blk[s], 0)),
        pl.BlockSpec((1, D, FB), lambda s, f, blk, eid: (eid[s], 0, f)),
        pl.BlockSpec((1, D, FB), lambda s, f, blk, eid: (eid[s], 0, f)),
        pl.BlockSpec((1, FB, D), lambda s, f, blk, eid: (eid[s], f, 0)),
    ],
    out_specs=pl.BlockSpec((B, D), lambda s, f, blk, eid: (blk[s], 0)),
)


def _routing_metadata(e2d):
    e_flat = e2d.reshape(P)
    oh = (e_flat[:, None] == jnp.arange(E, dtype=jnp.int32)[None, :])
    oh_b = oh.reshape(NW, P // NW, E).astype(jnp.float32)
    tril_fine = jnp.tril(jnp.ones((P // NW, P // NW), jnp.float32), k=-1)
    fine = jnp.einsum("ij,bjE->biE", tril_fine, oh_b,
                      precision=lax.Precision.HIGHEST)
    bs = oh_b.sum(axis=1)
    tril_coarse = jnp.tril(jnp.ones((NW, NW), jnp.float32), k=-1)
    coarse = tril_coarse @ bs
    rank = (fine + coarse[:, None, :]).reshape(P, E)
    rank = jnp.take_along_axis(rank, e_flat[:, None], axis=1)[:, 0]
    counts = bs.sum(axis=0)
    padded_counts = jnp.ceil(counts / B) * B
    pcsum = (jnp.tril(jnp.ones((E, E), jnp.float32)) @ padded_counts)
    pad_start = pcsum - padded_counts
    pos = (jnp.take(pad_start, e_flat) + rank).astype(jnp.int32)

    nb = (pcsum[E - 1] / B).astype(jnp.int32)
    s_ids = jnp.arange(G, dtype=jnp.int32)
    blk = jnp.minimum(s_ids, nb - 1)
    starts = (blk * B).astype(jnp.float32)
    eid = jnp.sum(pcsum[None, :] <= starts[:, None], axis=1).astype(jnp.int32)
    return pos.reshape(T, K), blk, eid


def kernel(x, w0, w1, w2, selected_experts, routing_weights):
    e2d = selected_experts.astype(jnp.int32)
    pos2d, blk, eid = _routing_metadata(e2d)

    x_padded = _make_sc_dispatch()(x, pos2d[:, 0], pos2d[:, 1])

    y = pl.pallas_call(
        _ffn_body,
        grid_spec=_ffn_grid_spec,
        out_shape=jax.ShapeDtypeStruct((P_PAD, D), jnp.float32),
    )(blk, eid, x_padded, w0, w1, w2)

    out = _make_sc_combine_tkd(32)(y, pos2d[:, 0], pos2d[:, 1])
    return out * routing_weights[:, :, None]


# speedup vs baseline: 1.4469x; 1.4469x over previous
import functools

import jax
import jax.numpy as jnp
from jax import lax
from jax.experimental import pallas as pl
from jax.experimental.pallas import tpu as pltpu
from jax.experimental.pallas import tpu_sc as plsc

T = 2048
D = 768
F = 2048
E = 8
K = 2

B = 256
P = T * K
P_PAD = P + E * B
G = P_PAD // B

NC = 2
NS = 16
NW = NC * NS
TPW = T // NW


def _sc_mesh():
    return plsc.VectorSubcoreMesh(core_axis_name="c", subcore_axis_name="s",
                                  num_cores=NC, num_subcores=NS)


@functools.lru_cache(maxsize=None)
def _make_sc_dispatch():
    @functools.partial(
        pl.kernel,
        mesh=_sc_mesh(),
        out_type=jax.ShapeDtypeStruct((P_PAD, D), jnp.float32),
        scratch_types=[
            pltpu.VMEM((TPW,), jnp.int32),
            pltpu.VMEM((TPW,), jnp.int32),
            pltpu.VMEM((TPW, D), jnp.float32),
            pltpu.SemaphoreType.DMA,
        ],
    )
    def dispatch_kernel(x_hbm, pe_hbm, po_hbm, out_hbm, idxe_v, idxo_v,
                        rows_v, sem):
        wid = lax.axis_index("s") * NC + lax.axis_index("c")
        tb = wid * TPW
        pltpu.sync_copy(x_hbm.at[pl.ds(tb, TPW)], rows_v)
        pltpu.sync_copy(pe_hbm.at[pl.ds(tb, TPW)], idxe_v)
        pltpu.sync_copy(po_hbm.at[pl.ds(tb, TPW)], idxo_v)
        c1 = pltpu.async_copy(rows_v, out_hbm.at[idxe_v], sem)
        c2 = pltpu.async_copy(rows_v, out_hbm.at[idxo_v], sem)
        c1.wait()
        c2.wait()

    return dispatch_kernel


@functools.lru_cache(maxsize=None)
def _make_sc_combine_tkd(chunk: int):
    n_per_w = T // NW
    n_chunks = n_per_w // chunk
    assert n_per_w % chunk == 0 and chunk % 8 == 0

    @functools.partial(
        pl.kernel,
        mesh=_sc_mesh(),
        out_type=jax.ShapeDtypeStruct((T, K, D), jnp.float32),
        scratch_types=[
            pltpu.VMEM((chunk,), jnp.int32),
            pltpu.VMEM((chunk,), jnp.int32),
            pltpu.VMEM((chunk, D), jnp.float32),
            pltpu.VMEM((chunk, D), jnp.float32),
            pltpu.SemaphoreType.DMA,
        ],
    )
    def combine_kernel(table_hbm, pe_hbm, po_hbm, out_hbm, idxe_v, idxo_v,
                       rows_a, rows_b, sem):
        wid = lax.axis_index("s") * NC + lax.axis_index("c")
        base = wid * n_per_w
        for c in range(n_chunks):
            tb = base + c * chunk
            pltpu.sync_copy(pe_hbm.at[pl.ds(tb, chunk)], idxe_v)
            pltpu.sync_copy(po_hbm.at[pl.ds(tb, chunk)], idxo_v)
            c1 = pltpu.async_copy(table_hbm.at[idxe_v], rows_a, sem)
            c2 = pltpu.async_copy(table_hbm.at[idxo_v], rows_b, sem)
            c1.wait()
            c2.wait()
            pltpu.sync_copy(rows_a, out_hbm.at[pl.ds(tb, chunk), 0])
            pltpu.sync_copy(rows_b, out_hbm.at[pl.ds(tb, chunk), 1])

    return combine_kernel


@functools.lru_cache(maxsize=None)
def _make_sc_combine(n_rows: int, chunk: int):
    n_per_w = n_rows // NW
    n_chunks = n_per_w // chunk
    assert n_per_w % chunk == 0 and chunk % 8 == 0

    @functools.partial(
        pl.kernel,
        mesh=_sc_mesh(),
        out_type=jax.ShapeDtypeStruct((n_rows, D), jnp.float32),
        scratch_types=[
            pltpu.VMEM((chunk,), jnp.int32),
            pltpu.VMEM((chunk, D), jnp.float32),
            pltpu.SemaphoreType.DMA,
        ],
    )
    def gather_kernel(table_hbm, idx_hbm, out_hbm, idx_v, rows_v, sem):
        wid = lax.axis_index("s") * NC + lax.axis_index("c")
        base = wid * n_per_w
        for c in range(n_chunks):
            off = base + c * chunk
            pltpu.sync_copy(idx_hbm.at[pl.ds(off, chunk)], idx_v)
            pltpu.async_copy(table_hbm.at[idx_v], rows_v, sem).wait()
            pltpu.sync_copy(rows_v, out_hbm.at[pl.ds(off, chunk)])

    return gather_kernel


def _ffn_body(blk_ref, eid_ref, x_ref, w0_ref, w1_ref, w2_ref, o_ref):
    s = pl.program_id(0)

    @pl.when(blk_ref[s] == s)
    def _():
        xb = x_ref[...]
        a = jnp.dot(xb, w0_ref[0], preferred_element_type=jnp.float32,
                    precision=lax.Precision.DEFAULT)
        b = jnp.dot(xb, w1_ref[0], preferred_element_type=jnp.float32,
                    precision=lax.Precision.DEFAULT)
        h = (a * jax.nn.sigmoid(a)) * b
        o_ref[...] = jnp.dot(h, w2_ref[0], preferred_element_type=jnp.float32,
                             precision=lax.Precision.DEFAULT)


_ffn_grid_spec = pltpu.PrefetchScalarGridSpec(
    num_scalar_prefetch=2,
    grid=(G,),
    in_specs=[
        pl.BlockSpec((B, D), lambda s, blk, eid: (blk[s], 0)),
        pl.BlockSpec((1, D, F), lambda s, blk, eid: (eid[s], 0, 0)),
        pl.BlockSpec((1, D, F), lambda s, blk, eid: (eid[s], 0, 0)),
        pl.BlockSpec((1, F, D), lambda s, blk, eid: (eid[s], 0, 0)),
    ],
    out_specs=pl.BlockSpec((B, D), lambda s, blk, eid: (blk[s], 0)),
)


def _routing_metadata(e2d):
    e_flat = e2d.reshape(P)
    oh = (e_flat[:, None] == jnp.arange(E, dtype=jnp.int32)[None, :])
    oh_b = oh.reshape(NW, P // NW, E).astype(jnp.float32)
    tril_fine = jnp.tril(jnp.ones((P // NW, P // NW), jnp.float32), k=-1)
    fine = jnp.einsum("ij,bjE->biE", tril_fine, oh_b,
                      precision=lax.Precision.HIGHEST)
    bs = oh_b.sum(axis=1)
    tril_coarse = jnp.tril(jnp.ones((NW, NW), jnp.float32), k=-1)
    coarse = tril_coarse @ bs
    rank = (fine + coarse[:, None, :]).reshape(P, E)
    rank = jnp.take_along_axis(rank, e_flat[:, None], axis=1)[:, 0]
    counts = bs.sum(axis=0)
    padded_counts = jnp.ceil(counts / B) * B
    pcsum = (jnp.tril(jnp.ones((E, E), jnp.float32)) @ padded_counts)
    pad_start = pcsum - padded_counts
    pos = (jnp.take(pad_start, e_flat) + rank).astype(jnp.int32)

    nb = (pcsum[E - 1] / B).astype(jnp.int32)
    s_ids = jnp.arange(G, dtype=jnp.int32)
    blk = jnp.minimum(s_ids, nb - 1)
    starts = (blk * B).astype(jnp.float32)
    eid = jnp.sum(pcsum[None, :] <= starts[:, None], axis=1).astype(jnp.int32)
    return pos.reshape(T, K), blk, eid


def kernel(x, w0, w1, w2, selected_experts, routing_weights):
    e2d = selected_experts.astype(jnp.int32)
    pos2d, blk, eid = _routing_metadata(e2d)

    x_padded = _make_sc_dispatch()(x, pos2d[:, 0], pos2d[:, 1])

    y = pl.pallas_call(
        _ffn_body,
        grid_spec=_ffn_grid_spec,
        out_shape=jax.ShapeDtypeStruct((P_PAD, D), jnp.float32),
    )(blk, eid, x_padded, w0, w1, w2)

    out = _make_sc_combine_tkd(32)(y, pos2d[:, 0], pos2d[:, 1])
    return out * routing_weights[:, :, None]


# speedup vs baseline: 1.5381x; 1.0630x over previous
import functools

import jax
import jax.numpy as jnp
from jax import lax
from jax.experimental import pallas as pl
from jax.experimental.pallas import tpu as pltpu
from jax.experimental.pallas import tpu_sc as plsc

T = 2048
D = 768
F = 2048
E = 8
K = 2

B = 256
P = T * K
P_PAD = P + E * B
G = P_PAD // B

NC = 2
NS = 16
NW = NC * NS
TPW = T // NW


def _sc_mesh():
    return plsc.VectorSubcoreMesh(core_axis_name="c", subcore_axis_name="s",
                                  num_cores=NC, num_subcores=NS)


@functools.lru_cache(maxsize=None)
def _make_sc_dispatch():
    @functools.partial(
        pl.kernel,
        mesh=_sc_mesh(),
        out_type=jax.ShapeDtypeStruct((P_PAD, D), jnp.float32),
        scratch_types=[
            pltpu.VMEM((TPW,), jnp.int32),
            pltpu.VMEM((TPW,), jnp.int32),
            pltpu.VMEM((TPW, D), jnp.float32),
            pltpu.SemaphoreType.DMA,
        ],
    )
    def dispatch_kernel(x_hbm, pe_hbm, po_hbm, out_hbm, idxe_v, idxo_v,
                        rows_v, sem):
        wid = lax.axis_index("s") * NC + lax.axis_index("c")
        tb = wid * TPW
        pltpu.sync_copy(x_hbm.at[pl.ds(tb, TPW)], rows_v)
        pltpu.sync_copy(pe_hbm.at[pl.ds(tb, TPW)], idxe_v)
        pltpu.sync_copy(po_hbm.at[pl.ds(tb, TPW)], idxo_v)
        c1 = pltpu.async_copy(rows_v, out_hbm.at[idxe_v], sem)
        c2 = pltpu.async_copy(rows_v, out_hbm.at[idxo_v], sem)
        c1.wait()
        c2.wait()

    return dispatch_kernel


@functools.lru_cache(maxsize=None)
def _make_sc_combine_tkd(chunk: int):
    n_per_w = T // NW
    n_chunks = n_per_w // chunk
    assert n_per_w % chunk == 0 and chunk % 8 == 0

    @functools.partial(
        pl.kernel,
        mesh=_sc_mesh(),
        out_type=jax.ShapeDtypeStruct((T, K, D), jnp.float32),
        scratch_types=[
            pltpu.VMEM((chunk,), jnp.int32),
            pltpu.VMEM((chunk,), jnp.int32),
            pltpu.VMEM((chunk, D), jnp.float32),
            pltpu.VMEM((chunk, D), jnp.float32),
            pltpu.SemaphoreType.DMA,
        ],
    )
    def combine_kernel(table_hbm, pe_hbm, po_hbm, out_hbm, idxe_v, idxo_v,
                       rows_a, rows_b, sem):
        wid = lax.axis_index("s") * NC + lax.axis_index("c")
        base = wid * n_per_w
        for c in range(n_chunks):
            tb = base + c * chunk
            pltpu.sync_copy(pe_hbm.at[pl.ds(tb, chunk)], idxe_v)
            pltpu.sync_copy(po_hbm.at[pl.ds(tb, chunk)], idxo_v)
            c1 = pltpu.async_copy(table_hbm.at[idxe_v], rows_a, sem)
            c2 = pltpu.async_copy(table_hbm.at[idxo_v], rows_b, sem)
            c1.wait()
            c2.wait()
            pltpu.sync_copy(rows_a, out_hbm.at[pl.ds(tb, chunk), 0])
            pltpu.sync_copy(rows_b, out_hbm.at[pl.ds(tb, chunk), 1])

    return combine_kernel


@functools.lru_cache(maxsize=None)
def _make_sc_combine(n_rows: int, chunk: int):
    n_per_w = n_rows // NW
    n_chunks = n_per_w // chunk
    assert n_per_w % chunk == 0 and chunk % 8 == 0

    @functools.partial(
        pl.kernel,
        mesh=_sc_mesh(),
        out_type=jax.ShapeDtypeStruct((n_rows, D), jnp.float32),
        scratch_types=[
            pltpu.VMEM((chunk,), jnp.int32),
            pltpu.VMEM((chunk, D), jnp.float32),
            pltpu.SemaphoreType.DMA,
        ],
    )
    def gather_kernel(table_hbm, idx_hbm, out_hbm, idx_v, rows_v, sem):
        wid = lax.axis_index("s") * NC + lax.axis_index("c")
        base = wid * n_per_w
        for c in range(n_chunks):
            off = base + c * chunk
            pltpu.sync_copy(idx_hbm.at[pl.ds(off, chunk)], idx_v)
            pltpu.async_copy(table_hbm.at[idx_v], rows_v, sem).wait()
            pltpu.sync_copy(rows_v, out_hbm.at[pl.ds(off, chunk)])

    return gather_kernel


def _ffn_body(meta_ref, x_ref, w0_ref, w1_ref, w2_ref, o_ref):
    s = pl.program_id(0)

    @pl.when(meta_ref[0, s] == s)
    def _():
        xb = x_ref[...]
        a = jnp.dot(xb, w0_ref[0], preferred_element_type=jnp.float32,
                    precision=lax.Precision.DEFAULT)
        b = jnp.dot(xb, w1_ref[0], preferred_element_type=jnp.float32,
                    precision=lax.Precision.DEFAULT)
        h = (a * jax.nn.sigmoid(a)) * b
        o_ref[...] = jnp.dot(h, w2_ref[0], preferred_element_type=jnp.float32,
                             precision=lax.Precision.DEFAULT)


_ffn_grid_spec = pltpu.PrefetchScalarGridSpec(
    num_scalar_prefetch=1,
    grid=(G,),
    in_specs=[
        pl.BlockSpec((B, D), lambda s, meta: (meta[0, s], 0)),
        pl.BlockSpec((1, D, F), lambda s, meta: (meta[1, s], 0, 0)),
        pl.BlockSpec((1, D, F), lambda s, meta: (meta[1, s], 0, 0)),
        pl.BlockSpec((1, F, D), lambda s, meta: (meta[1, s], 0, 0)),
    ],
    out_specs=pl.BlockSpec((B, D), lambda s, meta: (meta[0, s], 0)),
)


def _routing_metadata(e2d):
    e_flat = e2d.reshape(P)
    oh = (e_flat[:, None] == jnp.arange(E, dtype=jnp.int32)[None, :])
    oh_b = oh.reshape(NW, P // NW, E).astype(jnp.float32)
    tril_fine = jnp.tril(jnp.ones((P // NW, P // NW), jnp.float32), k=-1)
    fine = jnp.einsum("ij,bjE->biE", tril_fine, oh_b,
                      precision=lax.Precision.HIGHEST)
    bs = oh_b.sum(axis=1)
    tril_coarse = jnp.tril(jnp.ones((NW, NW), jnp.float32), k=-1)
    coarse = tril_coarse @ bs
    rank = (fine + coarse[:, None, :]).reshape(P, E)
    rank = jnp.take_along_axis(rank, e_flat[:, None], axis=1)[:, 0]
    counts = bs.sum(axis=0)
    padded_counts = jnp.ceil(counts / B) * B
    pcsum = (jnp.tril(jnp.ones((E, E), jnp.float32)) @ padded_counts)
    pad_start = pcsum - padded_counts
    pos = (jnp.take(pad_start, e_flat) + rank).astype(jnp.int32)

    nb = (pcsum[E - 1] / B).astype(jnp.int32)
    s_ids = jnp.arange(G, dtype=jnp.int32)
    blk = jnp.minimum(s_ids, nb - 1)
    starts = (blk * B).astype(jnp.float32)
    eid = jnp.sum(pcsum[None, :] <= starts[:, None], axis=1).astype(jnp.int32)
    return pos.reshape(T, K), blk, eid


NB_M = 32
LB = P // NB_M


def _meta_body(e_ref, pos_ref, meta_ref):
    ei = e_ref[...]
    ei_rows = jnp.repeat(ei, E, axis=0)
    r_i = lax.broadcasted_iota(jnp.int32, (NB_M * E, LB), 0)
    oh = (ei_rows == r_i % E).astype(jnp.float32)

    li = lax.broadcasted_iota(jnp.int32, (LB, LB), 0)
    lj = lax.broadcasted_iota(jnp.int32, (LB, LB), 1)
    triu_strict = (li < lj).astype(jnp.float32)
    fine = jnp.dot(oh, triu_strict,
                   preferred_element_type=jnp.float32)

    bs = jnp.sum(oh, axis=1, keepdims=True)
    ri = lax.broadcasted_iota(jnp.int32, (NB_M * E, NB_M * E), 0)
    rj = lax.broadcasted_iota(jnp.int32, (NB_M * E, NB_M * E), 1)
    same_e = (ri % E) == (rj % E)
    m_coarse = (same_e & ((rj // E) < (ri // E))).astype(jnp.float32)
    coarse = jnp.dot(m_coarse, bs,
                     preferred_element_type=jnp.float32)

    counts_r = jnp.dot(same_e.astype(jnp.float32), bs,
                       preferred_element_type=jnp.float32)
    padded_r = jnp.ceil(counts_r * (1.0 / B)) * B
    m_ps = (((rj % E) < (ri % E)) &
            ((rj // E) == (ri // E))).astype(jnp.float32)
    pad_start_r = jnp.dot(m_ps, padded_r,
                          preferred_element_type=jnp.float32)
    val = fine + coarse + pad_start_r
    sel = oh * val
    pos = jnp.sum(sel.reshape(NB_M, E, LB), axis=1).astype(jnp.int32)
    pos_ref[...] = pos

    pcsum_r = pad_start_r[:, :1] + padded_r
    pcsum_e = pcsum_r[:E, :]
    nb = (pcsum_e[E - 1:E, 0:1] / B).astype(jnp.int32)
    s_i = lax.broadcasted_iota(jnp.int32, (1, LB), 1)
    blk = jnp.minimum(s_i, nb - 1)
    starts = (blk * B).astype(jnp.float32)
    le = jnp.broadcast_to(pcsum_e, (E, LB))
    eid = jnp.sum((le <= starts).astype(jnp.int32), axis=0, keepdims=True)
    meta_ref[...] = jnp.concatenate([blk, eid], axis=0)


def _routing_metadata_pallas(e2d):
    pos, meta = pl.pallas_call(
        _meta_body,
        out_shape=(jax.ShapeDtypeStruct((NB_M, LB), jnp.int32),
                   jax.ShapeDtypeStruct((2, LB), jnp.int32)),
    )(e2d.reshape(NB_M, LB))
    return pos.reshape(T, K), meta


def kernel(x, w0, w1, w2, selected_experts, routing_weights):
    e2d = selected_experts.astype(jnp.int32)
    pos2d, meta = _routing_metadata_pallas(e2d)

    x_padded = _make_sc_dispatch()(x, pos2d[:, 0], pos2d[:, 1])

    y = pl.pallas_call(
        _ffn_body,
        grid_spec=_ffn_grid_spec,
        out_shape=jax.ShapeDtypeStruct((P_PAD, D), jnp.float32),
    )(meta, x_padded, w0, w1, w2)

    out = _make_sc_combine_tkd(32)(y, pos2d[:, 0], pos2d[:, 1])
    return out * routing_weights[:, :, None]


# speedup vs baseline: 1.6429x; 1.0681x over previous
import functools

import jax
import jax.numpy as jnp
from jax import lax
from jax.experimental import pallas as pl
from jax.experimental.pallas import tpu as pltpu
from jax.experimental.pallas import tpu_sc as plsc

T = 2048
D = 768
F = 2048
E = 8
K = 2

B = 512
P = T * K
P_PAD = P + E * B
G = P_PAD // B

NC = 2
NS = 16
NW = NC * NS
TPW = T // NW


def _sc_mesh():
    return plsc.VectorSubcoreMesh(core_axis_name="c", subcore_axis_name="s",
                                  num_cores=NC, num_subcores=NS)


@functools.lru_cache(maxsize=None)
def _make_sc_dispatch():
    @functools.partial(
        pl.kernel,
        mesh=_sc_mesh(),
        out_type=jax.ShapeDtypeStruct((P_PAD, D), jnp.float32),
        scratch_types=[
            pltpu.VMEM((TPW,), jnp.int32),
            pltpu.VMEM((TPW,), jnp.int32),
            pltpu.VMEM((TPW, D), jnp.float32),
            pltpu.SemaphoreType.DMA,
        ],
    )
    def dispatch_kernel(x_hbm, pe_hbm, po_hbm, out_hbm, idxe_v, idxo_v,
                        rows_v, sem):
        wid = lax.axis_index("s") * NC + lax.axis_index("c")
        tb = wid * TPW
        pltpu.sync_copy(x_hbm.at[pl.ds(tb, TPW)], rows_v)
        pltpu.sync_copy(pe_hbm.at[pl.ds(tb, TPW)], idxe_v)
        pltpu.sync_copy(po_hbm.at[pl.ds(tb, TPW)], idxo_v)
        c1 = pltpu.async_copy(rows_v, out_hbm.at[idxe_v], sem)
        c2 = pltpu.async_copy(rows_v, out_hbm.at[idxo_v], sem)
        c1.wait()
        c2.wait()

    return dispatch_kernel


@functools.lru_cache(maxsize=None)
def _make_sc_combine_tkd(chunk: int):
    n_per_w = T // NW
    n_chunks = n_per_w // chunk
    assert n_per_w % chunk == 0 and chunk % 8 == 0

    @functools.partial(
        pl.kernel,
        mesh=_sc_mesh(),
        out_type=jax.ShapeDtypeStruct((T, K, D), jnp.float32),
        scratch_types=[
            pltpu.VMEM((chunk,), jnp.int32),
            pltpu.VMEM((chunk,), jnp.int32),
            pltpu.VMEM((chunk, D), jnp.float32),
            pltpu.VMEM((chunk, D), jnp.float32),
            pltpu.SemaphoreType.DMA,
        ],
    )
    def combine_kernel(table_hbm, pe_hbm, po_hbm, out_hbm, idxe_v, idxo_v,
                       rows_a, rows_b, sem):
        wid = lax.axis_index("s") * NC + lax.axis_index("c")
        base = wid * n_per_w
        for c in range(n_chunks):
            tb = base + c * chunk
            pltpu.sync_copy(pe_hbm.at[pl.ds(tb, chunk)], idxe_v)
            pltpu.sync_copy(po_hbm.at[pl.ds(tb, chunk)], idxo_v)
            c1 = pltpu.async_copy(table_hbm.at[idxe_v], rows_a, sem)
            c2 = pltpu.async_copy(table_hbm.at[idxo_v], rows_b, sem)
            c1.wait()
            c2.wait()
            pltpu.sync_copy(rows_a, out_hbm.at[pl.ds(tb, chunk), 0])
            pltpu.sync_copy(rows_b, out_hbm.at[pl.ds(tb, chunk), 1])

    return combine_kernel


@functools.lru_cache(maxsize=None)
def _make_sc_combine(n_rows: int, chunk: int):
    n_per_w = n_rows // NW
    n_chunks = n_per_w // chunk
    assert n_per_w % chunk == 0 and chunk % 8 == 0

    @functools.partial(
        pl.kernel,
        mesh=_sc_mesh(),
        out_type=jax.ShapeDtypeStruct((n_rows, D), jnp.float32),
        scratch_types=[
            pltpu.VMEM((chunk,), jnp.int32),
            pltpu.VMEM((chunk, D), jnp.float32),
            pltpu.SemaphoreType.DMA,
        ],
    )
    def gather_kernel(table_hbm, idx_hbm, out_hbm, idx_v, rows_v, sem):
        wid = lax.axis_index("s") * NC + lax.axis_index("c")
        base = wid * n_per_w
        for c in range(n_chunks):
            off = base + c * chunk
            pltpu.sync_copy(idx_hbm.at[pl.ds(off, chunk)], idx_v)
            pltpu.async_copy(table_hbm.at[idx_v], rows_v, sem).wait()
            pltpu.sync_copy(rows_v, out_hbm.at[pl.ds(off, chunk)])

    return gather_kernel


def _ffn_body(meta_ref, x_ref, w0_ref, w1_ref, w2_ref, o_ref):
    s = pl.program_id(0)

    @pl.when(meta_ref[0, s] == s)
    def _():
        xb = x_ref[...]
        a = jnp.dot(xb, w0_ref[0], preferred_element_type=jnp.float32,
                    precision=lax.Precision.DEFAULT)
        b = jnp.dot(xb, w1_ref[0], preferred_element_type=jnp.float32,
                    precision=lax.Precision.DEFAULT)
        h = (a * jax.nn.sigmoid(a)) * b
        o_ref[...] = jnp.dot(h, w2_ref[0], preferred_element_type=jnp.float32,
                             precision=lax.Precision.DEFAULT)


_ffn_grid_spec = pltpu.PrefetchScalarGridSpec(
    num_scalar_prefetch=1,
    grid=(G,),
    in_specs=[
        pl.BlockSpec((B, D), lambda s, meta: (meta[0, s], 0)),
        pl.BlockSpec((1, D, F), lambda s, meta: (meta[1, s], 0, 0)),
        pl.BlockSpec((1, D, F), lambda s, meta: (meta[1, s], 0, 0)),
        pl.BlockSpec((1, F, D), lambda s, meta: (meta[1, s], 0, 0)),
    ],
    out_specs=pl.BlockSpec((B, D), lambda s, meta: (meta[0, s], 0)),
)


def _routing_metadata(e2d):
    e_flat = e2d.reshape(P)
    oh = (e_flat[:, None] == jnp.arange(E, dtype=jnp.int32)[None, :])
    oh_b = oh.reshape(NW, P // NW, E).astype(jnp.float32)
    tril_fine = jnp.tril(jnp.ones((P // NW, P // NW), jnp.float32), k=-1)
    fine = jnp.einsum("ij,bjE->biE", tril_fine, oh_b,
                      precision=lax.Precision.HIGHEST)
    bs = oh_b.sum(axis=1)
    tril_coarse = jnp.tril(jnp.ones((NW, NW), jnp.float32), k=-1)
    coarse = tril_coarse @ bs
    rank = (fine + coarse[:, None, :]).reshape(P, E)
    rank = jnp.take_along_axis(rank, e_flat[:, None], axis=1)[:, 0]
    counts = bs.sum(axis=0)
    padded_counts = jnp.ceil(counts / B) * B
    pcsum = (jnp.tril(jnp.ones((E, E), jnp.float32)) @ padded_counts)
    pad_start = pcsum - padded_counts
    pos = (jnp.take(pad_start, e_flat) + rank).astype(jnp.int32)

    nb = (pcsum[E - 1] / B).astype(jnp.int32)
    s_ids = jnp.arange(G, dtype=jnp.int32)
    blk = jnp.minimum(s_ids, nb - 1)
    starts = (blk * B).astype(jnp.float32)
    eid = jnp.sum(pcsum[None, :] <= starts[:, None], axis=1).astype(jnp.int32)
    return pos.reshape(T, K), blk, eid


NB_M = 32
LB = P // NB_M


def _meta_body(e_ref, pos_ref, meta_ref):
    ei = e_ref[...]
    ei_rows = jnp.repeat(ei, E, axis=0)
    r_i = lax.broadcasted_iota(jnp.int32, (NB_M * E, LB), 0)
    oh = (ei_rows == r_i % E).astype(jnp.float32)

    li = lax.broadcasted_iota(jnp.int32, (LB, LB), 0)
    lj = lax.broadcasted_iota(jnp.int32, (LB, LB), 1)
    triu_strict = (li < lj).astype(jnp.float32)
    fine = jnp.dot(oh, triu_strict,
                   preferred_element_type=jnp.float32)

    bs = jnp.sum(oh, axis=1, keepdims=True)
    ri = lax.broadcasted_iota(jnp.int32, (NB_M * E, NB_M * E), 0)
    rj = lax.broadcasted_iota(jnp.int32, (NB_M * E, NB_M * E), 1)
    same_e = (ri % E) == (rj % E)
    m_coarse = (same_e & ((rj // E) < (ri // E))).astype(jnp.float32)
    coarse = jnp.dot(m_coarse, bs,
                     preferred_element_type=jnp.float32)

    counts_r = jnp.dot(same_e.astype(jnp.float32), bs,
                       preferred_element_type=jnp.float32)
    padded_r = jnp.ceil(counts_r * (1.0 / B)) * B
    m_ps = (((rj % E) < (ri % E)) &
            ((rj // E) == (ri // E))).astype(jnp.float32)
    pad_start_r = jnp.dot(m_ps, padded_r,
                          preferred_element_type=jnp.float32)
    val = fine + coarse + pad_start_r
    sel = oh * val
    pos = jnp.sum(sel.reshape(NB_M, E, LB), axis=1).astype(jnp.int32)
    pos_ref[...] = pos

    pcsum_r = pad_start_r[:, :1] + padded_r
    pcsum_e = pcsum_r[:E, :]
    nb = (pcsum_e[E - 1:E, 0:1] / B).astype(jnp.int32)
    s_i = lax.broadcasted_iota(jnp.int32, (1, LB), 1)
    blk = jnp.minimum(s_i, nb - 1)
    starts = (blk * B).astype(jnp.float32)
    le = jnp.broadcast_to(pcsum_e, (E, LB))
    eid = jnp.sum((le <= starts).astype(jnp.int32), axis=0, keepdims=True)
    meta_ref[...] = jnp.concatenate([blk, eid], axis=0)


def _routing_metadata_pallas(e2d):
    pos, meta = pl.pallas_call(
        _meta_body,
        out_shape=(jax.ShapeDtypeStruct((NB_M, LB), jnp.int32),
                   jax.ShapeDtypeStruct((2, LB), jnp.int32)),
    )(e2d.reshape(NB_M, LB))
    return pos.reshape(T, K), meta


def kernel(x, w0, w1, w2, selected_experts, routing_weights):
    e2d = selected_experts.astype(jnp.int32)
    pos2d, meta = _routing_metadata_pallas(e2d)

    x_padded = _make_sc_dispatch()(x, pos2d[:, 0], pos2d[:, 1])

    y = pl.pallas_call(
        _ffn_body,
        grid_spec=_ffn_grid_spec,
        out_shape=jax.ShapeDtypeStruct((P_PAD, D), jnp.float32),
    )(meta, x_padded, w0, w1, w2)

    out = _make_sc_combine_tkd(32)(y, pos2d[:, 0], pos2d[:, 1])
    return out * routing_weights[:, :, None]
